# baseline (speedup 1.0000x reference)
import functools

import jax
import jax.numpy as jnp
from jax import lax
from jax.experimental import pallas as pl
from jax.experimental.pallas import tpu as pltpu
from jax.experimental.pallas import tpu_sc as plsc

_NC = 2
_NS = 16
_NW = _NC * _NS
_LANES = 16

_CARD = 1000
_D_TOK = 16


def _sc_gather(emb_t, xcat_flat, b, out_cols, col0):
    d, v = emb_t.shape
    total = xcat_flat.shape[0]
    halves = _NW // d
    seg = total // halves
    shift = b.bit_length() - 1
    assert halves * d == _NW and seg * halves == total and b == (1 << shift)

    mesh = plsc.VectorSubcoreMesh(
        core_axis_name="c", subcore_axis_name="s",
        num_cores=_NC, num_subcores=_NS,
    )

    @functools.partial(
        pl.kernel,
        out_type=jax.ShapeDtypeStruct((d, out_cols), jnp.float32),
        mesh=mesh,
        scratch_types=[
            pltpu.VMEM((v,), jnp.float32),
            pltpu.VMEM((seg,), jnp.int32),
            pltpu.VMEM((seg,), jnp.float32),
        ],
        compiler_params=pltpu.CompilerParams(use_tc_tiling_on_sc=False,
                                             needs_layout_passes=False),
    )
    def gather_kernel(emb_hbm, idx_hbm, out_hbm, plane_v, idx_v, res_v):
        wid = lax.axis_index("s") * _NC + lax.axis_index("c")
        p = lax.rem(wid, d)
        base = (wid // d) * seg
        pltpu.sync_copy(emb_hbm.at[p], plane_v)
        pltpu.sync_copy(idx_hbm.at[pl.ds(base, seg)], idx_v)
        iota = lax.broadcasted_iota(jnp.int32, (_LANES,), 0)

        unroll = 4

        def body(i, carry):
            for u in range(unroll):
                off = (i * unroll + u) * _LANES
                field = lax.shift_right_logical(base + off + iota, shift)
                gidx = idx_v[pl.ds(off, _LANES)] + field * _CARD
                res_v[pl.ds(off, _LANES)] = plsc.load_gather(plane_v, [gidx])
            return carry

        lax.fori_loop(0, seg // (_LANES * unroll), body, 0)
        pltpu.sync_copy(res_v, out_hbm.at[p, pl.ds(col0 + base, seg)])

    return gather_kernel(emb_t, xcat_flat)


def _tc_body(n_cat, xnumt_ref, g_ref, tokw_ref, tokb_ref, recw_ref,
             rlw_ref, rlb_ref, onumt_ref, ocat_ref):
    f32 = jnp.float32
    bf16 = jnp.bfloat16
    d_num = xnumt_ref.shape[0]
    n = pl.program_id(0)

    @pl.when(n == 0)
    def _():
        recw = recw_ref[...]
        a = jnp.sum(tokw_ref[0:d_num, :] * recw, axis=1)
        bias13 = jnp.concatenate(
            [jnp.zeros((1, _D_TOK), f32), tokb_ref[0:d_num - 1, :]], axis=0)
        c = jnp.sum(bias13 * recw, axis=1)
        rows = lax.broadcasted_iota(jnp.int32, (d_num, d_num), 0)
        cols = lax.broadcasted_iota(jnp.int32, (d_num, d_num), 1)
        m2 = jnp.where(cols == rows - 1, a[:, None], jnp.zeros((), f32))
        c2 = c + jnp.where(
            lax.broadcasted_iota(jnp.int32, (d_num,), 0) == 0, a[0], 0.0)
        onumt_ref[...] = lax.dot_general(
            m2, xnumt_ref[...], (((1,), (0,)), ((), ())),
            preferred_element_type=f32) + c2[:, None]

    h0t = tokw_ref[d_num, :][:, None] * xnumt_ref[d_num - 1:d_num, :]
    for t in range(2):
        fld = 2 * n + t
        wt = rlw_ref[t].astype(bf16)
        rlb_col = rlb_ref[t, 0, :]
        tb = tokb_ref[d_num - 1 + fld, :]
        gt = g_ref[:, t * 1024:(t + 1) * 1024]
        ht = jnp.where(fld == 0, h0t, gt) + tb[:, None]
        out = lax.dot_general(wt, ht.astype(bf16), (((0,), (0,)), ((), ())),
                              preferred_element_type=f32)
        ocat_ref[t] = out + rlb_col[:, None]


def kernel(x_num, x_cat, tok_weight, tok_bias, cat_emb, category_offsets,
           rec_weight, rec_lin_w, rec_lin_b):
    b, d_num = x_num.shape
    n_cat = x_cat.shape[1]
    n_used = n_cat - 1
    card = rec_lin_w.shape[1]
    out_cols = (n_cat + 2) * b
    col0 = 3 * b

    xcat_flat = x_cat[:, :n_used].T.reshape(n_used * b)
    g2 = _sc_gather(cat_emb.T, xcat_flat, b, out_cols, col0)
    xnumt = x_num.T
    rlwt = rec_lin_w.transpose(0, 2, 1)
    rlb3 = rec_lin_b.reshape(n_cat, 1, card)

    grid = (n_cat // 2,)
    onumt, ocat_p = pl.pallas_call(
        functools.partial(_tc_body, n_cat),
        grid=grid,
        in_specs=[
            pl.BlockSpec(xnumt.shape, lambda n: (0, 0)),
            pl.BlockSpec((_D_TOK, 2 * b), lambda n: (0, n + 1)),
            pl.BlockSpec(tok_weight.shape, lambda n: (0, 0)),
            pl.BlockSpec(tok_bias.shape, lambda n: (0, 0)),
            pl.BlockSpec(rec_weight.shape, lambda n: (0, 0)),
            pl.BlockSpec((2, _D_TOK, card), lambda n: (n, 0, 0)),
            pl.BlockSpec((2, 1, card), lambda n: (n, 0, 0)),
        ],
        out_specs=[
            pl.BlockSpec(xnumt.shape, lambda n: (0, 0)),
            pl.BlockSpec((2, card, b), lambda n: (n, 0, 0)),
        ],
        out_shape=[
            jax.ShapeDtypeStruct((d_num, b), jnp.float32),
            jax.ShapeDtypeStruct((n_cat, card, b), jnp.float32),
        ],
    )(xnumt, g2, tok_weight, tok_bias, rec_weight, rlwt, rlb3)
    return onumt.T, ocat_p.transpose(2, 0, 1)
